# baseline (speedup 1.0000x reference)
import functools

import jax
import jax.numpy as jnp
from jax import lax
from jax.experimental import pallas as pl
from jax.experimental.pallas import tpu as pltpu
from jax.experimental.pallas import tpu_sc as plsc


def kernel(categorical_features, emb_tables):
    B, F = categorical_features.shape
    Ft, V, D = emb_tables.shape
    assert Ft == F

    flat_table = emb_tables.reshape(F * V, D)
    flat_idx = categorical_features.reshape(B * F).astype(jnp.int32)

    info = plsc.get_sparse_core_info()
    NC, NS, L = info.num_cores, info.num_subcores, info.num_lanes
    NW = NC * NS
    n = B * F
    assert n % NW == 0
    n_per_w = n // NW
    assert n_per_w % L == 0 and n_per_w % F == 0

    mesh = plsc.VectorSubcoreMesh(core_axis_name="c", subcore_axis_name="s")

    @functools.partial(
        pl.kernel,
        mesh=mesh,
        out_type=jax.ShapeDtypeStruct((n, D), jnp.float32),
        scratch_types=[
            pltpu.VMEM((n_per_w,), jnp.int32),
            pltpu.VMEM((n_per_w, D), jnp.float32),
            pltpu.SemaphoreType.DMA,
        ],
        compiler_params=pltpu.CompilerParams(use_tc_tiling_on_sc=False),
    )
    def sc_gather(idx_hbm, table_hbm, out_hbm, idx_v, rows_v, sem):
        wid = lax.axis_index("s") * NC + lax.axis_index("c")
        base = wid * n_per_w
        pltpu.sync_copy(idx_hbm.at[pl.ds(base, n_per_w)], idx_v)

        lane = lax.iota(jnp.int32, L)

        def add_offsets(i, carry):
            p = lane + i * L
            offs = (p % F) * V
            idx_v[pl.ds(i * L, L)] = idx_v[pl.ds(i * L, L)] + offs
            return carry

        lax.fori_loop(0, n_per_w // L, add_offsets, 0)

        pltpu.async_copy(table_hbm.at[idx_v], rows_v, sem).wait()
        pltpu.sync_copy(rows_v, out_hbm.at[pl.ds(base, n_per_w)])

    out = sc_gather(flat_idx, flat_table)
    return out.reshape(B, F * D)


# speedup vs baseline: 2.9039x; 2.9039x over previous
import functools

import jax
import jax.numpy as jnp
from jax import lax
from jax.experimental import pallas as pl
from jax.experimental.pallas import tpu as pltpu
from jax.experimental.pallas import tpu_sc as plsc


def kernel(categorical_features, emb_tables):
    B, F = categorical_features.shape
    Ft, V, D = emb_tables.shape
    assert Ft == F

    tab1 = emb_tables.transpose(2, 0, 1).reshape(D * F * V)
    idx1 = categorical_features.astype(jnp.int32).T.reshape(F * B)

    info = plsc.get_sparse_core_info()
    NC, NS, L = info.num_cores, info.num_subcores, info.num_lanes
    NW = NC * NS
    NT = F * D
    BH = 8192
    assert B % BH == 0 and BH % L == 0

    mesh = plsc.VectorSubcoreMesh(core_axis_name="c", subcore_axis_name="s")

    @functools.partial(
        pl.kernel,
        mesh=mesh,
        out_type=jax.ShapeDtypeStruct((NT * B,), jnp.float32),
        scratch_types=[
            pltpu.VMEM((B,), jnp.int32),
            pltpu.VMEM((V,), jnp.float32),
            pltpu.VMEM((BH,), jnp.float32),
        ],
        compiler_params=pltpu.CompilerParams(
            use_tc_tiling_on_sc=False, needs_layout_passes=False
        ),
    )
    def sc_lookup(idx_hbm, tab_hbm, out_hbm, idx_v, row_v, out_v):
        wid = lax.axis_index("s") * NC + lax.axis_index("c")
        lo = (wid * NT) // NW
        hi = ((wid + 1) * NT) // NW

        def task(t, carry):
            f = t // D
            d = t - f * D
            pltpu.sync_copy(idx_hbm.at[pl.ds(f * B, B)], idx_v)
            pltpu.sync_copy(tab_hbm.at[pl.ds((d * F + f) * V, V)], row_v)

            for h in range(B // BH):

                def gather_vec(j, c):
                    vec_idx = idx_v[pl.ds(h * BH + j * L, L)]
                    out_v[pl.ds(j * L, L)] = plsc.load_gather(row_v, [vec_idx])
                    return c

                lax.fori_loop(0, BH // L, gather_vec, 0)
                pltpu.sync_copy(out_v, out_hbm.at[pl.ds(t * B + h * BH, BH)])
            return carry

        lax.fori_loop(lo, hi, task, 0)

    out = sc_lookup(idx1, tab1)
    return out.reshape(NT, B).T


# speedup vs baseline: 25.1853x; 8.6728x over previous
import functools

import jax
import jax.numpy as jnp
from jax import lax
from jax.experimental import pallas as pl
from jax.experimental.pallas import tpu as pltpu
from jax.experimental.pallas import tpu_sc as plsc


def kernel(categorical_features, emb_tables):
    B, F = categorical_features.shape
    Ft, V, D = emb_tables.shape
    assert Ft == F

    tabT = emb_tables.transpose(2, 0, 1)
    idxT = categorical_features.astype(jnp.int32).T

    info = plsc.get_sparse_core_info()
    NC, NS, L = info.num_cores, info.num_subcores, info.num_lanes
    NW = NC * NS
    NT = F * D
    BH = 8192
    assert B % BH == 0 and BH % L == 0

    mesh = plsc.VectorSubcoreMesh(core_axis_name="c", subcore_axis_name="s")

    @functools.partial(
        pl.kernel,
        mesh=mesh,
        out_type=jax.ShapeDtypeStruct((NT, B), jnp.float32),
        scratch_types=[
            pltpu.VMEM((B,), jnp.int32),
            pltpu.VMEM((V,), jnp.float32),
            pltpu.VMEM((BH,), jnp.float32),
        ],
        compiler_params=pltpu.CompilerParams(needs_layout_passes=False),
    )
    def sc_lookup(idx_hbm, tab_hbm, out_hbm, idx_v, row_v, out_v):
        wid = lax.axis_index("s") * NC + lax.axis_index("c")
        lo = (wid * NT) // NW
        hi = ((wid + 1) * NT) // NW

        def task(t, carry):
            f = t // D
            d = t - f * D
            pltpu.sync_copy(idx_hbm.at[f], idx_v)
            pltpu.sync_copy(tab_hbm.at[d, f], row_v)

            for h in range(B // BH):

                def gather_vec(j, c):
                    vec_idx = idx_v[pl.ds(h * BH + j * L, L)]
                    out_v[pl.ds(j * L, L)] = plsc.load_gather(row_v, [vec_idx])
                    return c

                lax.fori_loop(0, BH // L, gather_vec, 0)
                pltpu.sync_copy(out_v, out_hbm.at[t, pl.ds(h * BH, BH)])
            return carry

        lax.fori_loop(lo, hi, task, 0)

    out = sc_lookup(idxT, tabT)
    return out.T


# speedup vs baseline: 43.7029x; 1.7353x over previous
import functools

import jax
import jax.numpy as jnp
from jax import lax
from jax.experimental import pallas as pl
from jax.experimental.pallas import tpu as pltpu
from jax.experimental.pallas import tpu_sc as plsc


def kernel(categorical_features, emb_tables):
    B, F = categorical_features.shape
    Ft, V, D = emb_tables.shape
    assert Ft == F

    tabT = emb_tables.transpose(2, 0, 1)
    idxT = categorical_features.astype(jnp.int32).T

    info = plsc.get_sparse_core_info()
    NC, NS, L = info.num_cores, info.num_subcores, info.num_lanes
    NW = NC * NS
    NT = F * D
    BH = 4096
    NH = B // BH
    assert B % BH == 0 and BH % L == 0 and NH >= 2

    mesh = plsc.VectorSubcoreMesh(core_axis_name="c", subcore_axis_name="s")

    @functools.partial(
        pl.kernel,
        mesh=mesh,
        out_type=jax.ShapeDtypeStruct((NT, B), jnp.float32),
        scratch_types=[
            pltpu.VMEM((B,), jnp.int32),
            pltpu.VMEM((V,), jnp.float32),
            pltpu.VMEM((BH,), jnp.float32),
            pltpu.VMEM((BH,), jnp.float32),
            pltpu.SemaphoreType.DMA,
            pltpu.SemaphoreType.DMA,
        ],
        compiler_params=pltpu.CompilerParams(needs_layout_passes=False),
    )
    def sc_lookup(
        idx_hbm, tab_hbm, out_hbm, idx_v, row_v, out_v0, out_v1, sem_t, sem_o
    ):
        wid = lax.axis_index("s") * NC + lax.axis_index("c")
        lo = (wid * NT) // NW
        hi = ((wid + 1) * NT) // NW

        def task(t, prev_f):
            f = t // D
            d = t - f * D
            lane_cp = pltpu.async_copy(tab_hbm.at[d, f], row_v, sem_t)

            @pl.when(f != prev_f)
            def _():
                pltpu.sync_copy(idx_hbm.at[f], idx_v)

            lane_cp.wait()

            out_cps = []
            for h in range(NH):
                if h >= 2:
                    out_cps[h - 2].wait()
                buf = out_v0 if h % 2 == 0 else out_v1

                @plsc.parallel_loop(0, BH // L, unroll=8)
                def gather_vec(j):
                    vec_idx = idx_v[pl.ds(h * BH + j * L, L)]
                    buf[pl.ds(j * L, L)] = plsc.load_gather(row_v, [vec_idx])

                out_cps.append(
                    pltpu.async_copy(
                        buf, out_hbm.at[t, pl.ds(h * BH, BH)], sem_o
                    )
                )
            out_cps[NH - 2].wait()
            out_cps[NH - 1].wait()
            return f

        lax.fori_loop(lo, hi, task, -1)

    out = sc_lookup(idxT, tabT)
    return out.T


# speedup vs baseline: 44.1704x; 1.0107x over previous
import functools

import jax
import jax.numpy as jnp
from jax import lax
from jax.experimental import pallas as pl
from jax.experimental.pallas import tpu as pltpu
from jax.experimental.pallas import tpu_sc as plsc


def kernel(categorical_features, emb_tables):
    B, F = categorical_features.shape
    Ft, V, D = emb_tables.shape
    assert Ft == F

    tabT = emb_tables.transpose(2, 0, 1)
    idxT = categorical_features.astype(jnp.int32).T

    info = plsc.get_sparse_core_info()
    NC, NS, L = info.num_cores, info.num_subcores, info.num_lanes
    NW = NC * NS
    NT = F * D
    BH = 4096
    NH = B // BH
    assert B % BH == 0 and BH % L == 0 and NH >= 2

    mesh = plsc.VectorSubcoreMesh(core_axis_name="c", subcore_axis_name="s")

    @functools.partial(
        pl.kernel,
        mesh=mesh,
        out_type=jax.ShapeDtypeStruct((NT, B), jnp.float32),
        scratch_types=[
            pltpu.VMEM((B,), jnp.int32),
            pltpu.VMEM((V,), jnp.float32),
            pltpu.VMEM((BH,), jnp.float32),
            pltpu.VMEM((BH,), jnp.float32),
            pltpu.SemaphoreType.DMA,
            pltpu.SemaphoreType.DMA,
        ],
        compiler_params=pltpu.CompilerParams(needs_layout_passes=False),
    )
    def sc_lookup(
        idx_hbm, tab_hbm, out_hbm, idx_v, row_v, out_v0, out_v1, sem_t, sem_o
    ):
        wid = lax.axis_index("s") * NC + lax.axis_index("c")
        lo = (wid * NT) // NW
        hi = ((wid + 1) * NT) // NW

        def task(t, prev_f):
            f = t // D
            d = t - f * D
            lane_cp = pltpu.async_copy(tab_hbm.at[d, f], row_v, sem_t)

            @pl.when(f != prev_f)
            def _():
                pltpu.sync_copy(idx_hbm.at[f], idx_v)

            lane_cp.wait()

            for h in range(NH):
                buf = out_v0 if h % 2 == 0 else out_v1
                g = (t - lo) * NH + h

                @pl.when(g >= 2)
                def _():
                    pltpu.make_async_copy(
                        buf, out_hbm.at[t, pl.ds(h * BH, BH)], sem_o
                    ).wait()

                @plsc.parallel_loop(0, BH // L, unroll=16)
                def gather_vec(j):
                    vec_idx = idx_v[pl.ds(h * BH + j * L, L)]
                    buf[pl.ds(j * L, L)] = plsc.load_gather(row_v, [vec_idx])

                pltpu.async_copy(buf, out_hbm.at[t, pl.ds(h * BH, BH)], sem_o)
            return f

        lax.fori_loop(lo, hi, task, -1)

        @pl.when(hi > lo)
        def _():
            for _ in range(2):
                pltpu.make_async_copy(
                    out_v0, out_hbm.at[0, pl.ds(0, BH)], sem_o
                ).wait()

    out = sc_lookup(idxT, tabT)
    return out.T
